# baseline (speedup 1.0000x reference)
import functools

import jax
import jax.numpy as jnp
from jax.experimental import pallas as pl
from jax.experimental.pallas import tpu as pltpu

B = 16
N = 32768
IN_CH = 256
ATT = 128
NEG_INF = -1e30

_DN_T = (((1,), (1,)), ((), ()))


def _to_col(row):
    ii = jax.lax.broadcasted_iota(jnp.int32, (B, B), 0)
    jj = jax.lax.broadcasted_iota(jnp.int32, (B, B), 1)
    return jnp.sum(jnp.where(ii == jj, jnp.broadcast_to(row, (B, B)), 0),
                   axis=1, keepdims=True)


def _half(xh, base, hb, lo, hi, w1_ref, b1_ref, w2_ref, b2_ref, ws_ref,
          bs_ref):
    rv = jax.lax.dot_general(xh, w1_ref[...], _DN_T,
                             preferred_element_type=jnp.float32) + b1_ref[...]
    ru = jax.lax.dot_general(xh, w2_ref[...], _DN_T,
                             preferred_element_type=jnp.float32) + b2_ref[...]
    g = jnp.tanh(rv) * jax.nn.sigmoid(ru)
    score = jax.lax.dot_general(ws_ref[...], g, _DN_T,
                                preferred_element_type=jnp.float32) + bs_ref[...]
    pos = base + jax.lax.broadcasted_iota(jnp.int32, (1, hb), 1)
    onehot = (pos >= lo) & (pos < hi)
    return score, onehot


def _fused_kernel(idx_ref, xa_ref, xb_ref, w1_ref, b1_ref, w2_ref, b2_ref,
                  ws_ref, bs_ref, wm1_ref, bm1_ref, wm2_ref, bm2_ref,
                  out_ref, m_ref, s_ref, acc_ref, *, hb, nblocks):
    i = pl.program_id(0)

    @pl.when(i == 0)
    def _init():
        m_ref[...] = jnp.full((B, 1), NEG_INF, jnp.float32)
        s_ref[...] = jnp.zeros((B, 1), jnp.float32)
        acc_ref[...] = jnp.zeros((B, IN_CH), jnp.float32)

    hi = _to_col(idx_ref[...])
    lo = jnp.concatenate([jnp.zeros((1, 1), jnp.int32), hi[:B - 1, :]], axis=0)

    xa = xa_ref[...]
    xb = xb_ref[...]
    score_a, oh_a = _half(xa, (2 * i) * hb, hb, lo, hi, w1_ref, b1_ref,
                          w2_ref, b2_ref, ws_ref, bs_ref)
    score_b, oh_b = _half(xb, (2 * i + 1) * hb, hb, lo, hi, w1_ref, b1_ref,
                          w2_ref, b2_ref, ws_ref, bs_ref)

    m_old = m_ref[...]
    sm_a = jnp.max(jnp.where(oh_a, score_a, NEG_INF), axis=1, keepdims=True)
    sm_b = jnp.max(jnp.where(oh_b, score_b, NEG_INF), axis=1, keepdims=True)
    m_new = jnp.maximum(m_old, jnp.maximum(sm_a, sm_b))
    scale = jnp.exp(m_old - m_new)

    mt_a = jnp.max(jnp.where(oh_a, m_new, NEG_INF), axis=0, keepdims=True)
    mt_b = jnp.max(jnp.where(oh_b, m_new, NEG_INF), axis=0, keepdims=True)
    e_a = jnp.where(oh_a, jnp.exp(score_a - mt_a), 0.0)
    e_b = jnp.where(oh_b, jnp.exp(score_b - mt_b), 0.0)

    s_ref[...] = (s_ref[...] * scale
                  + jnp.sum(e_a, axis=1, keepdims=True)
                  + jnp.sum(e_b, axis=1, keepdims=True))
    dn_pool = (((1,), (0,)), ((), ()))
    acc_ref[...] = (acc_ref[...] * scale
                    + jax.lax.dot_general(e_a, xa, dn_pool,
                                          preferred_element_type=jnp.float32)
                    + jax.lax.dot_general(e_b, xb, dn_pool,
                                          preferred_element_type=jnp.float32))
    m_ref[...] = m_new

    @pl.when(i == nblocks - 1)
    def _finish():
        slice_x = acc_ref[...] / s_ref[...]
        h = jax.nn.relu(jax.lax.dot_general(
            slice_x, wm1_ref[...], _DN_T,
            preferred_element_type=jnp.float32) + bm1_ref[...])
        out_ref[...] = jax.nn.relu(jax.lax.dot_general(
            wm2_ref[...], h, _DN_T,
            preferred_element_type=jnp.float32) + bm2_ref[...])


@functools.partial(jax.jit, static_argnames=("tb",))
def _run(x, idx, W1, b1, W2, b2, Ws, bs, Wm1, bm1, Wm2, bm2, tb=8192):
    nblocks = N // tb
    hb = tb // 2

    in_specs = [
        pl.BlockSpec((1, B), lambda i: (0, 0)),
        pl.BlockSpec((hb, IN_CH), lambda i: (2 * i, 0)),
        pl.BlockSpec((hb, IN_CH), lambda i: (2 * i + 1, 0)),
        pl.BlockSpec((ATT, IN_CH), lambda i: (0, 0)),
        pl.BlockSpec((1, ATT), lambda i: (0, 0)),
        pl.BlockSpec((ATT, IN_CH), lambda i: (0, 0)),
        pl.BlockSpec((1, ATT), lambda i: (0, 0)),
        pl.BlockSpec((1, ATT), lambda i: (0, 0)),
        pl.BlockSpec((1, 1), lambda i: (0, 0)),
        pl.BlockSpec((IN_CH // 8, IN_CH), lambda i: (0, 0)),
        pl.BlockSpec((1, IN_CH // 8), lambda i: (0, 0)),
        pl.BlockSpec((1, IN_CH // 8), lambda i: (0, 0)),
        pl.BlockSpec((1, 1), lambda i: (0, 0)),
    ]

    pred_row = pl.pallas_call(
        functools.partial(_fused_kernel, hb=hb, nblocks=nblocks),
        grid=(nblocks,),
        in_specs=in_specs,
        out_specs=pl.BlockSpec((1, B), lambda i: (0, 0)),
        out_shape=jax.ShapeDtypeStruct((1, B), jnp.float32),
        scratch_shapes=[
            pltpu.VMEM((B, 1), jnp.float32),
            pltpu.VMEM((B, 1), jnp.float32),
            pltpu.VMEM((B, IN_CH), jnp.float32),
        ],
        compiler_params=pltpu.CompilerParams(
            dimension_semantics=("arbitrary",),
        ),
    )(idx.reshape(1, B), x, x, W1, b1.reshape(1, ATT), W2,
      b2.reshape(1, ATT), Ws, bs.reshape(1, 1), Wm1,
      bm1.reshape(1, IN_CH // 8), Wm2, bm2.reshape(1, 1))
    return pred_row.reshape(B, 1)


def kernel(x, idx, W1, b1, W2, b2, Ws, bs, Wm1, bm1, Wm2, bm2):
    return _run(x, idx, W1, b1, W2, b2, Ws, bs, Wm1, bm1, Wm2, bm2)


# speedup vs baseline: 1.1229x; 1.1229x over previous
import functools

import jax
import jax.numpy as jnp
from jax.experimental import pallas as pl
from jax.experimental.pallas import tpu as pltpu

B = 16
N = 32768
IN_CH = 256
ATT = 128
NEG_INF = -1e30

_DN_T = (((1,), (1,)), ((), ()))


def _to_col(row):
    ii = jax.lax.broadcasted_iota(jnp.int32, (B, B), 0)
    jj = jax.lax.broadcasted_iota(jnp.int32, (B, B), 1)
    return jnp.sum(jnp.where(ii == jj, jnp.broadcast_to(row, (B, B)), 0),
                   axis=1, keepdims=True)


def _half(xh, base, hb, lo, hi, wc_ref, bc_ref, ws_ref, bs_ref):
    r = jax.lax.dot_general(xh.astype(jnp.bfloat16), wc_ref[...], _DN_T,
                            preferred_element_type=jnp.float32) + bc_ref[...]
    g = jnp.tanh(r[:, :ATT]) * jax.nn.sigmoid(r[:, ATT:])
    score = jax.lax.dot_general(ws_ref[...], g, _DN_T,
                                preferred_element_type=jnp.float32) + bs_ref[...]
    pos = base + jax.lax.broadcasted_iota(jnp.int32, (1, hb), 1)
    onehot = (pos >= lo) & (pos < hi)
    return score, onehot


def _fused_kernel(idx_ref, xa_ref, xb_ref, w1_ref, b1_ref, w2_ref, b2_ref,
                  ws_ref, bs_ref, wm1_ref, bm1_ref, wm2_ref, bm2_ref,
                  out_ref, m_ref, s_ref, acc_ref, wc_ref, bc_ref,
                  *, hb, nblocks):
    i = pl.program_id(0)

    @pl.when(i == 0)
    def _init():
        m_ref[...] = jnp.full((B, 1), NEG_INF, jnp.float32)
        s_ref[...] = jnp.zeros((B, 1), jnp.float32)
        acc_ref[...] = jnp.zeros((B, IN_CH), jnp.float32)
        wc_ref[:ATT, :] = w1_ref[...].astype(jnp.bfloat16)
        wc_ref[ATT:, :] = w2_ref[...].astype(jnp.bfloat16)
        bc_ref[:, :ATT] = b1_ref[...]
        bc_ref[:, ATT:] = b2_ref[...]

    hi = _to_col(idx_ref[...])
    lo = jnp.concatenate([jnp.zeros((1, 1), jnp.int32), hi[:B - 1, :]], axis=0)

    xa = xa_ref[...]
    xb = xb_ref[...]
    score_a, oh_a = _half(xa, (2 * i) * hb, hb, lo, hi, wc_ref, bc_ref,
                          ws_ref, bs_ref)
    score_b, oh_b = _half(xb, (2 * i + 1) * hb, hb, lo, hi, wc_ref, bc_ref,
                          ws_ref, bs_ref)

    m_old = m_ref[...]
    sm_a = jnp.max(jnp.where(oh_a, score_a, NEG_INF), axis=1, keepdims=True)
    sm_b = jnp.max(jnp.where(oh_b, score_b, NEG_INF), axis=1, keepdims=True)
    m_new = jnp.maximum(m_old, jnp.maximum(sm_a, sm_b))
    scale = jnp.exp(m_old - m_new)

    mt_a = jnp.max(jnp.where(oh_a, m_new, NEG_INF), axis=0, keepdims=True)
    mt_b = jnp.max(jnp.where(oh_b, m_new, NEG_INF), axis=0, keepdims=True)
    e_a = jnp.where(oh_a, jnp.exp(score_a - mt_a), 0.0)
    e_b = jnp.where(oh_b, jnp.exp(score_b - mt_b), 0.0)

    s_ref[...] = (s_ref[...] * scale
                  + jnp.sum(e_a, axis=1, keepdims=True)
                  + jnp.sum(e_b, axis=1, keepdims=True))
    dn_pool = (((1,), (0,)), ((), ()))
    acc_ref[...] = (acc_ref[...] * scale
                    + jax.lax.dot_general(e_a, xa, dn_pool,
                                          preferred_element_type=jnp.float32)
                    + jax.lax.dot_general(e_b, xb, dn_pool,
                                          preferred_element_type=jnp.float32))
    m_ref[...] = m_new

    @pl.when(i == nblocks - 1)
    def _finish():
        slice_x = acc_ref[...] / s_ref[...]
        h = jax.nn.relu(jax.lax.dot_general(
            slice_x, wm1_ref[...], _DN_T,
            preferred_element_type=jnp.float32) + bm1_ref[...])
        out_ref[...] = jax.nn.relu(jax.lax.dot_general(
            wm2_ref[...], h, _DN_T,
            preferred_element_type=jnp.float32) + bm2_ref[...])


@functools.partial(jax.jit, static_argnames=("tb",))
def _run(x, idx, W1, b1, W2, b2, Ws, bs, Wm1, bm1, Wm2, bm2, tb=8192):
    nblocks = N // tb
    hb = tb // 2

    in_specs = [
        pl.BlockSpec((1, B), lambda i: (0, 0)),
        pl.BlockSpec((hb, IN_CH), lambda i: (2 * i, 0)),
        pl.BlockSpec((hb, IN_CH), lambda i: (2 * i + 1, 0)),
        pl.BlockSpec((ATT, IN_CH), lambda i: (0, 0)),
        pl.BlockSpec((1, ATT), lambda i: (0, 0)),
        pl.BlockSpec((ATT, IN_CH), lambda i: (0, 0)),
        pl.BlockSpec((1, ATT), lambda i: (0, 0)),
        pl.BlockSpec((1, ATT), lambda i: (0, 0)),
        pl.BlockSpec((1, 1), lambda i: (0, 0)),
        pl.BlockSpec((IN_CH // 8, IN_CH), lambda i: (0, 0)),
        pl.BlockSpec((1, IN_CH // 8), lambda i: (0, 0)),
        pl.BlockSpec((1, IN_CH // 8), lambda i: (0, 0)),
        pl.BlockSpec((1, 1), lambda i: (0, 0)),
    ]

    pred_row = pl.pallas_call(
        functools.partial(_fused_kernel, hb=hb, nblocks=nblocks),
        grid=(nblocks,),
        in_specs=in_specs,
        out_specs=pl.BlockSpec((1, B), lambda i: (0, 0)),
        out_shape=jax.ShapeDtypeStruct((1, B), jnp.float32),
        scratch_shapes=[
            pltpu.VMEM((B, 1), jnp.float32),
            pltpu.VMEM((B, 1), jnp.float32),
            pltpu.VMEM((B, IN_CH), jnp.float32),
            pltpu.VMEM((2 * ATT, IN_CH), jnp.bfloat16),
            pltpu.VMEM((1, 2 * ATT), jnp.float32),
        ],
        compiler_params=pltpu.CompilerParams(
            dimension_semantics=("arbitrary",),
        ),
    )(idx.reshape(1, B), x, x, W1, b1.reshape(1, ATT), W2,
      b2.reshape(1, ATT), Ws, bs.reshape(1, 1), Wm1,
      bm1.reshape(1, IN_CH // 8), Wm2, bm2.reshape(1, 1))
    return pred_row.reshape(B, 1)


def kernel(x, idx, W1, b1, W2, b2, Ws, bs, Wm1, bm1, Wm2, bm2):
    return _run(x, idx, W1, b1, W2, b2, Ws, bs, Wm1, bm1, Wm2, bm2)
